# Initial kernel scaffold; baseline (speedup 1.0000x reference)
import jax
import jax.numpy as jnp
from jax.experimental import pallas as pl


def kernel(x, emb, W_p, b_p):
    raise NotImplementedError("write your pallas kernel here")


# speedup vs baseline: 1.9584x; 1.9584x over previous
import functools

import jax
import jax.numpy as jnp
from jax.experimental import pallas as pl
from jax.experimental.pallas import tpu as pltpu

_IN_DIM = 768
_LATENT = 64
_NT0, _NT1 = 32, 32
_NEMB = _NT0 * _NT1
_COMMIT = 0.32
_SOM_MULT = 1.2

_BLK = 512


def _proj_body(x_ref, wT_ref, b_ref, ze_ref):
    ze_ref[...] = jnp.dot(x_ref[...], wT_ref[...]) + b_ref[...]


def _vq_body(ze_ref, embT_ref, emb_ref,
             zq_ref, k_ref, loss_ref, *, nrows):
    i = pl.program_id(0)

    z_e = ze_ref[...]
    embT = embT_ref[...]
    s = jnp.dot(z_e, embT)
    ze2 = jnp.sum(z_e * z_e, axis=1, keepdims=True)
    e2 = jnp.sum(embT * embT, axis=0, keepdims=True)
    dist = jnp.sqrt(jnp.maximum((ze2 + e2) - 2.0 * s, 0.0))

    m = jnp.min(dist, axis=1, keepdims=True)
    ci = jax.lax.broadcasted_iota(jnp.int32, (_BLK, _NEMB), 1)
    kk = jnp.min(jnp.where(dist == m, ci, _NEMB), axis=1, keepdims=True)
    k_ref[0, 0, :] = kk[:, 0]

    k1 = kk >> 5
    k2 = kk & 31
    m0 = ci == kk
    mu = (ci == kk + _NT1) & (k1 < _NT0 - 1)
    md = (ci == kk - _NT1) & (k1 > 0)
    mr = (ci == kk + 1) & (k2 < _NT1 - 1)
    ml = (ci == kk - 1) & (k2 > 0)
    mall = m0 | mu | md | mr | ml

    zero = jnp.zeros_like(s)
    e2b = jnp.broadcast_to(e2, s.shape)
    s_k = jnp.sum(jnp.where(m0, s, zero), axis=1, keepdims=True)
    e2_k = jnp.sum(jnp.where(m0, e2b, zero), axis=1, keepdims=True)
    nsdot = jnp.sum(jnp.where(mall, s, zero), axis=1, keepdims=True)
    nq = jnp.sum(jnp.where(mall, e2b, zero), axis=1, keepdims=True)

    commit_part = jnp.sum(ze2 - 2.0 * s_k + e2_k)
    som_part = jnp.sum(5.0 * ze2 - 2.0 * nsdot + nq)
    c1 = _COMMIT / (nrows * _LATENT)
    c2 = _SOM_MULT / (nrows * 5 * _LATENT)
    part = c1 * commit_part + c2 * som_part

    @pl.when(i == 0)
    def _():
        loss_ref[0, 0] = 0.0

    loss_ref[0, 0] += part

    onehot = m0.astype(jnp.float32)
    zq_ref[...] = jnp.dot(onehot, emb_ref[...],
                          precision=jax.lax.Precision.HIGHEST)


def kernel(x, emb, W_p, b_p):
    inp_shape = x.shape[:-1]
    n = x.shape[0] * x.shape[1]
    xf = x.reshape(n, _IN_DIM)
    nblk = n // _BLK

    wT = W_p.T
    b2 = b_p.reshape(1, _LATENT)
    embT = emb.T

    z_e = pl.pallas_call(
        _proj_body,
        grid=(nblk,),
        in_specs=[
            pl.BlockSpec((_BLK, _IN_DIM), lambda i: (i, 0)),
            pl.BlockSpec((_IN_DIM, _LATENT), lambda i: (0, 0)),
            pl.BlockSpec((1, _LATENT), lambda i: (0, 0)),
        ],
        out_specs=pl.BlockSpec((_BLK, _LATENT), lambda i: (i, 0)),
        out_shape=jax.ShapeDtypeStruct((n, _LATENT), jnp.float32),
    )(xf, wT, b2)

    z_q, k3, loss = pl.pallas_call(
        functools.partial(_vq_body, nrows=n),
        grid=(nblk,),
        in_specs=[
            pl.BlockSpec((_BLK, _LATENT), lambda i: (i, 0)),
            pl.BlockSpec((_LATENT, _NEMB), lambda i: (0, 0)),
            pl.BlockSpec((_NEMB, _LATENT), lambda i: (0, 0)),
        ],
        out_specs=[
            pl.BlockSpec((_BLK, _LATENT), lambda i: (i, 0)),
            pl.BlockSpec((1, 1, _BLK), lambda i: (i, 0, 0)),
            pl.BlockSpec(memory_space=pltpu.SMEM),
        ],
        out_shape=[
            jax.ShapeDtypeStruct((n, _LATENT), jnp.float32),
            jax.ShapeDtypeStruct((nblk, 1, _BLK), jnp.int32),
            jax.ShapeDtypeStruct((1, 1), jnp.float32),
        ],
    )(z_e, embT, emb)

    z_q_out = z_q.reshape(inp_shape + (_LATENT,))
    z_e_out = z_e.reshape(inp_shape + (_LATENT,))
    k_out = k3.reshape(inp_shape)
    return (z_q_out, loss[0, 0], z_e_out, z_q_out, k_out[..., None])


# speedup vs baseline: 2.1072x; 1.0760x over previous
import functools

import jax
import jax.numpy as jnp
from jax import lax
from jax.experimental import pallas as pl
from jax.experimental.pallas import tpu as pltpu
from jax.experimental.pallas import tpu_sc as plsc

_IN_DIM = 768
_LATENT = 64
_NT0, _NT1 = 32, 32
_NEMB = _NT0 * _NT1
_COMMIT = 0.32
_SOM_MULT = 1.2

_BLK = 512


def _proj_body(x_ref, wT_ref, b_ref, ze_ref):
    ze_ref[...] = jnp.dot(x_ref[...], wT_ref[...]) + b_ref[...]


def _vq_body(ze_ref, embT_ref,
             k_ref, loss_ref, *, nrows):
    i = pl.program_id(0)

    z_e = ze_ref[...]
    embT = embT_ref[...]
    s = jnp.dot(z_e, embT)
    ze2 = jnp.sum(z_e * z_e, axis=1, keepdims=True)
    e2 = jnp.sum(embT * embT, axis=0, keepdims=True)
    dist = jnp.sqrt(jnp.maximum((ze2 + e2) - 2.0 * s, 0.0))

    m = jnp.min(dist, axis=1, keepdims=True)
    ci = jax.lax.broadcasted_iota(jnp.int32, (_BLK, _NEMB), 1)
    kk = jnp.min(jnp.where(dist == m, ci, _NEMB), axis=1, keepdims=True)
    k_ref[0, 0, :] = kk[:, 0]

    k1 = kk >> 5
    k2 = kk & 31
    m0 = ci == kk
    mu = (ci == kk + _NT1) & (k1 < _NT0 - 1)
    md = (ci == kk - _NT1) & (k1 > 0)
    mr = (ci == kk + 1) & (k2 < _NT1 - 1)
    ml = (ci == kk - 1) & (k2 > 0)
    mall = m0 | mu | md | mr | ml

    zero = jnp.zeros_like(s)
    e2b = jnp.broadcast_to(e2, s.shape)
    s_k = jnp.sum(jnp.where(m0, s, zero), axis=1, keepdims=True)
    e2_k = jnp.sum(jnp.where(m0, e2b, zero), axis=1, keepdims=True)
    nsdot = jnp.sum(jnp.where(mall, s, zero), axis=1, keepdims=True)
    nq = jnp.sum(jnp.where(mall, e2b, zero), axis=1, keepdims=True)

    commit_part = jnp.sum(ze2 - 2.0 * s_k + e2_k)
    som_part = jnp.sum(5.0 * ze2 - 2.0 * nsdot + nq)
    c1 = _COMMIT / (nrows * _LATENT)
    c2 = _SOM_MULT / (nrows * 5 * _LATENT)
    part = c1 * commit_part + c2 * som_part

    @pl.when(i == 0)
    def _():
        loss_ref[0, 0] = 0.0

    loss_ref[0, 0] += part


_GCHUNK = 96


def _sc_gather_rows(emb_pad, idx, n):
    info = plsc.get_sparse_core_info()
    nc, ns = info.num_cores, info.num_subcores
    nw = nc * ns
    b_per_w = n // nw
    width = emb_pad.shape[1]
    mesh = plsc.VectorSubcoreMesh(core_axis_name="c", subcore_axis_name="s")

    @functools.partial(
        pl.kernel, mesh=mesh,
        out_type=jax.ShapeDtypeStruct((n, width), jnp.float32),
        scratch_types=[
            pltpu.VMEM((b_per_w,), jnp.int32),
            pltpu.VMEM((b_per_w, width), jnp.float32),
            pltpu.SemaphoreType.DMA,
        ],
    )
    def _gather(emb_hbm, idx_hbm, out_hbm, idx_v, rows_v, sem):
        wid = lax.axis_index("s") * nc + lax.axis_index("c")
        base = wid * b_per_w
        pltpu.sync_copy(idx_hbm.at[pl.ds(base, b_per_w)], idx_v)
        descs = []
        for off in range(0, b_per_w, _GCHUNK):
            descs.append(pltpu.async_copy(
                emb_hbm.at[idx_v.at[pl.ds(off, _GCHUNK)]],
                rows_v.at[pl.ds(off, _GCHUNK)], sem))
        for d in descs:
            d.wait()
        pltpu.sync_copy(rows_v, out_hbm.at[pl.ds(base, b_per_w)])

    return _gather(emb_pad, idx)


def kernel(x, emb, W_p, b_p):
    inp_shape = x.shape[:-1]
    n = x.shape[0] * x.shape[1]
    xf = x.reshape(n, _IN_DIM)
    nblk = n // _BLK

    wT = W_p.T
    b2 = b_p.reshape(1, _LATENT)
    embT = emb.T

    z_e = pl.pallas_call(
        _proj_body,
        grid=(nblk,),
        in_specs=[
            pl.BlockSpec((_BLK, _IN_DIM), lambda i: (i, 0)),
            pl.BlockSpec((_IN_DIM, _LATENT), lambda i: (0, 0)),
            pl.BlockSpec((1, _LATENT), lambda i: (0, 0)),
        ],
        out_specs=pl.BlockSpec((_BLK, _LATENT), lambda i: (i, 0)),
        out_shape=jax.ShapeDtypeStruct((n, _LATENT), jnp.float32),
    )(xf, wT, b2)

    k3, loss = pl.pallas_call(
        functools.partial(_vq_body, nrows=n),
        grid=(nblk,),
        in_specs=[
            pl.BlockSpec((_BLK, _LATENT), lambda i: (i, 0)),
            pl.BlockSpec((_LATENT, _NEMB), lambda i: (0, 0)),
        ],
        out_specs=[
            pl.BlockSpec((1, 1, _BLK), lambda i: (i, 0, 0)),
            pl.BlockSpec(memory_space=pltpu.SMEM),
        ],
        out_shape=[
            jax.ShapeDtypeStruct((nblk, 1, _BLK), jnp.int32),
            jax.ShapeDtypeStruct((1, 1), jnp.float32),
        ],
    )(z_e, embT)

    emb_pad = jnp.pad(emb, ((0, 0), (0, 128 - _LATENT)))
    z_q = _sc_gather_rows(emb_pad, k3.reshape(n), n)[:, :_LATENT]

    z_q_out = z_q.reshape(inp_shape + (_LATENT,))
    z_e_out = z_e.reshape(inp_shape + (_LATENT,))
    k_out = k3.reshape(inp_shape)
    return (z_q_out, loss[0, 0], z_e_out, z_q_out, k_out[..., None])


# speedup vs baseline: 2.7513x; 1.3057x over previous
import functools

import jax
import jax.numpy as jnp
from jax import lax
from jax.experimental import pallas as pl
from jax.experimental.pallas import tpu as pltpu
from jax.experimental.pallas import tpu_sc as plsc

_IN_DIM = 768
_LATENT = 64
_NT0, _NT1 = 32, 32
_NEMB = _NT0 * _NT1
_COMMIT = 0.32
_SOM_MULT = 1.2

_BLK = 512


def _proj_body(x_ref, wT_ref, b_ref, ze_ref):
    ze_ref[...] = jnp.dot(x_ref[...], wT_ref[...]) + b_ref[...]


def _vq_body(ze_ref, embT_ref,
             k_ref, loss_ref, *, nrows):
    i = pl.program_id(0)

    z_e = ze_ref[...]
    embT = embT_ref[...]
    s = jnp.dot(z_e, embT)
    ze2 = jnp.sum(z_e * z_e, axis=1, keepdims=True)
    e2 = jnp.sum(embT * embT, axis=0, keepdims=True)
    dist = jnp.sqrt(jnp.maximum((ze2 + e2) - 2.0 * s, 0.0))

    m = jnp.min(dist, axis=1, keepdims=True)
    ci = jax.lax.broadcasted_iota(jnp.int32, (_BLK, _NEMB), 1)
    kk = jnp.min(jnp.where(dist == m, ci, _NEMB), axis=1, keepdims=True)
    k_ref[0, 0, :] = kk[:, 0]

    k1 = kk >> 5
    k2 = kk & 31
    none = jnp.full_like(kk, -1)
    tu = jnp.where(k1 < _NT0 - 1, kk + _NT1, none)
    td = jnp.where(k1 > 0, kk - _NT1, none)
    tr = jnp.where(k2 < _NT1 - 1, kk + 1, none)
    tl = jnp.where(k2 > 0, kk - 1, none)
    mall = (ci == kk) | (ci == tu) | (ci == td) | (ci == tr) | (ci == tl)

    dist2 = dist * dist
    nbsum = jnp.sum(jnp.where(mall, dist2, jnp.zeros_like(dist2)),
                    axis=1, keepdims=True)
    cnt = (1
           + (k1 < _NT0 - 1).astype(jnp.float32)
           + (k1 > 0).astype(jnp.float32)
           + (k2 < _NT1 - 1).astype(jnp.float32)
           + (k2 > 0).astype(jnp.float32))
    commit_part = jnp.sum(m * m)
    som_part = jnp.sum(nbsum + (5.0 - cnt) * ze2)
    c1 = _COMMIT / (nrows * _LATENT)
    c2 = _SOM_MULT / (nrows * 5 * _LATENT)
    part = c1 * commit_part + c2 * som_part

    @pl.when(i == 0)
    def _():
        loss_ref[0, 0] = 0.0

    loss_ref[0, 0] += part


_GCHUNK = 96


def _sc_gather_rows(emb_pad, idx, n):
    info = plsc.get_sparse_core_info()
    nc, ns = info.num_cores, info.num_subcores
    nw = nc * ns
    b_per_w = n // nw
    width = emb_pad.shape[1]
    mesh = plsc.VectorSubcoreMesh(core_axis_name="c", subcore_axis_name="s")

    @functools.partial(
        pl.kernel, mesh=mesh,
        out_type=jax.ShapeDtypeStruct((n, width), jnp.float32),
        scratch_types=[
            pltpu.VMEM((b_per_w,), jnp.int32),
            pltpu.VMEM((b_per_w, width), jnp.float32),
            pltpu.SemaphoreType.DMA,
        ],
    )
    def _gather(emb_hbm, idx_hbm, out_hbm, idx_v, rows_v, sem):
        wid = lax.axis_index("s") * nc + lax.axis_index("c")
        base = wid * b_per_w
        pltpu.sync_copy(idx_hbm.at[pl.ds(base, b_per_w)], idx_v)
        descs = []
        for off in range(0, b_per_w, _GCHUNK):
            descs.append(pltpu.async_copy(
                emb_hbm.at[idx_v.at[pl.ds(off, _GCHUNK)]],
                rows_v.at[pl.ds(off, _GCHUNK)], sem))
        for d in descs:
            d.wait()
        pltpu.sync_copy(rows_v, out_hbm.at[pl.ds(base, b_per_w)])

    return _gather(emb_pad, idx)


def kernel(x, emb, W_p, b_p):
    inp_shape = x.shape[:-1]
    n = x.shape[0] * x.shape[1]
    xf = x.reshape(n, _IN_DIM)
    nblk = n // _BLK

    wT = W_p.T
    b2 = b_p.reshape(1, _LATENT)
    embT = emb.T

    z_e = pl.pallas_call(
        _proj_body,
        grid=(nblk,),
        in_specs=[
            pl.BlockSpec((_BLK, _IN_DIM), lambda i: (i, 0)),
            pl.BlockSpec((_IN_DIM, _LATENT), lambda i: (0, 0)),
            pl.BlockSpec((1, _LATENT), lambda i: (0, 0)),
        ],
        out_specs=pl.BlockSpec((_BLK, _LATENT), lambda i: (i, 0)),
        out_shape=jax.ShapeDtypeStruct((n, _LATENT), jnp.float32),
    )(xf, wT, b2)

    k3, loss = pl.pallas_call(
        functools.partial(_vq_body, nrows=n),
        grid=(nblk,),
        in_specs=[
            pl.BlockSpec((_BLK, _LATENT), lambda i: (i, 0)),
            pl.BlockSpec((_LATENT, _NEMB), lambda i: (0, 0)),
        ],
        out_specs=[
            pl.BlockSpec((1, 1, _BLK), lambda i: (i, 0, 0)),
            pl.BlockSpec(memory_space=pltpu.SMEM),
        ],
        out_shape=[
            jax.ShapeDtypeStruct((nblk, 1, _BLK), jnp.int32),
            jax.ShapeDtypeStruct((1, 1), jnp.float32),
        ],
    )(z_e, embT)

    emb_pad = jnp.pad(emb, ((0, 0), (0, 128 - _LATENT)))
    z_q = _sc_gather_rows(emb_pad, k3.reshape(n), n)[:, :_LATENT]

    z_q_out = z_q.reshape(inp_shape + (_LATENT,))
    z_e_out = z_e.reshape(inp_shape + (_LATENT,))
    k_out = k3.reshape(inp_shape)
    return (z_q_out, loss[0, 0], z_e_out, z_q_out, k_out[..., None])


# speedup vs baseline: 2.9180x; 1.0606x over previous
import functools

import jax
import jax.numpy as jnp
from jax import lax
from jax.experimental import pallas as pl
from jax.experimental.pallas import tpu as pltpu
from jax.experimental.pallas import tpu_sc as plsc

_IN_DIM = 768
_LATENT = 64
_NT0, _NT1 = 32, 32
_NEMB = _NT0 * _NT1
_COMMIT = 0.32
_SOM_MULT = 1.2

_BLKA = 1024
_BLKB = 512


def _proj_body(x_ref, wT_ref, b_ref, ze_ref):
    ze_ref[...] = jnp.dot(x_ref[...], wT_ref[...]) + b_ref[...]


def _vq_body(ze_ref, embT_ref,
             k_ref, loss_ref, *, nrows):
    i = pl.program_id(0)

    z_e = ze_ref[...]
    embT = embT_ref[...]
    s = jnp.dot(z_e, embT)
    ze2 = jnp.sum(z_e * z_e, axis=1, keepdims=True)
    e2 = jnp.sum(embT * embT, axis=0, keepdims=True)
    dist = jnp.sqrt(jnp.maximum((ze2 + e2) - 2.0 * s, 0.0))

    m = jnp.min(dist, axis=1, keepdims=True)
    ci = jax.lax.broadcasted_iota(jnp.int32, (_BLKB, _NEMB), 1)
    kk = jnp.min(jnp.where(dist == m, ci, _NEMB), axis=1, keepdims=True)
    k_ref[0, 0, :] = kk[:, 0]

    k1 = kk >> 5
    k2 = kk & 31
    none = jnp.full_like(kk, -1)
    tu = jnp.where(k1 < _NT0 - 1, kk + _NT1, none)
    td = jnp.where(k1 > 0, kk - _NT1, none)
    tr = jnp.where(k2 < _NT1 - 1, kk + 1, none)
    tl = jnp.where(k2 > 0, kk - 1, none)
    mall = (ci == kk) | (ci == tu) | (ci == td) | (ci == tr) | (ci == tl)

    dist2 = dist * dist
    nbsum = jnp.sum(jnp.where(mall, dist2, jnp.zeros_like(dist2)),
                    axis=1, keepdims=True)
    cnt = (1
           + (k1 < _NT0 - 1).astype(jnp.float32)
           + (k1 > 0).astype(jnp.float32)
           + (k2 < _NT1 - 1).astype(jnp.float32)
           + (k2 > 0).astype(jnp.float32))
    commit_part = jnp.sum(m * m)
    som_part = jnp.sum(nbsum + (5.0 - cnt) * ze2)
    c1 = _COMMIT / (nrows * _LATENT)
    c2 = _SOM_MULT / (nrows * 5 * _LATENT)
    part = c1 * commit_part + c2 * som_part

    @pl.when(i == 0)
    def _():
        loss_ref[0, 0] = 0.0

    loss_ref[0, 0] += part


_GCHUNK = 96


def _sc_gather_rows(emb_pad, idx, n):
    info = plsc.get_sparse_core_info()
    nc, ns = info.num_cores, info.num_subcores
    nw = nc * ns
    b_per_w = n // nw
    width = emb_pad.shape[1]
    mesh = plsc.VectorSubcoreMesh(core_axis_name="c", subcore_axis_name="s")

    @functools.partial(
        pl.kernel, mesh=mesh,
        out_type=jax.ShapeDtypeStruct((n, width), jnp.float32),
        scratch_types=[
            pltpu.VMEM((b_per_w,), jnp.int32),
            pltpu.VMEM((b_per_w, width), jnp.float32),
            pltpu.SemaphoreType.DMA,
        ],
    )
    def _gather(emb_hbm, idx_hbm, out_hbm, idx_v, rows_v, sem):
        wid = lax.axis_index("s") * nc + lax.axis_index("c")
        base = wid * b_per_w
        pltpu.sync_copy(idx_hbm.at[pl.ds(base, b_per_w)], idx_v)
        descs = []
        for off in range(0, b_per_w, _GCHUNK):
            descs.append(pltpu.async_copy(
                emb_hbm.at[idx_v.at[pl.ds(off, _GCHUNK)]],
                rows_v.at[pl.ds(off, _GCHUNK)], sem))
        for d in descs:
            d.wait()
        pltpu.sync_copy(rows_v, out_hbm.at[pl.ds(base, b_per_w)])

    return _gather(emb_pad, idx)


def kernel(x, emb, W_p, b_p):
    inp_shape = x.shape[:-1]
    n = x.shape[0] * x.shape[1]
    xf = x.reshape(n, _IN_DIM)
    nblk = n // _BLKB

    wT = W_p.T
    b2 = b_p.reshape(1, _LATENT)
    embT = emb.T

    z_e = pl.pallas_call(
        _proj_body,
        grid=(n // _BLKA,),
        in_specs=[
            pl.BlockSpec((_BLKA, _IN_DIM), lambda i: (i, 0)),
            pl.BlockSpec((_IN_DIM, _LATENT), lambda i: (0, 0)),
            pl.BlockSpec((1, _LATENT), lambda i: (0, 0)),
        ],
        out_specs=pl.BlockSpec((_BLKA, _LATENT), lambda i: (i, 0)),
        out_shape=jax.ShapeDtypeStruct((n, _LATENT), jnp.float32),
    )(xf, wT, b2)

    k3, loss = pl.pallas_call(
        functools.partial(_vq_body, nrows=n),
        grid=(nblk,),
        in_specs=[
            pl.BlockSpec((_BLKB, _LATENT), lambda i: (i, 0)),
            pl.BlockSpec((_LATENT, _NEMB), lambda i: (0, 0)),
        ],
        out_specs=[
            pl.BlockSpec((1, 1, _BLKB), lambda i: (i, 0, 0)),
            pl.BlockSpec(memory_space=pltpu.SMEM),
        ],
        out_shape=[
            jax.ShapeDtypeStruct((nblk, 1, _BLKB), jnp.int32),
            jax.ShapeDtypeStruct((1, 1), jnp.float32),
        ],
    )(z_e, embT)

    emb_pad = jnp.pad(emb, ((0, 0), (0, 128 - _LATENT)))
    z_q = _sc_gather_rows(emb_pad, k3.reshape(n), n)[:, :_LATENT]

    z_q_out = z_q.reshape(inp_shape + (_LATENT,))
    z_e_out = z_e.reshape(inp_shape + (_LATENT,))
    k_out = k3.reshape(inp_shape)
    return (z_q_out, loss[0, 0], z_e_out, z_q_out, k_out[..., None])
